# baseline (speedup 1.0000x reference)
import jax
import jax.numpy as jnp
from jax import lax
from jax.experimental import pallas as pl
from jax.experimental.pallas import tpu as pltpu
from jax.experimental.pallas import tpu_sc as plsc

P = 602112
C = 33
F = 40
NSUM = 34
LEVEL = 6
RES = 2 ** LEVEL
M = RES ** 3
CONTRACT_SCALE = 4.0
DENSITY_CLAMP = 6.0

NC = 2
NS = 16
WIN = 40960
GARB = 512
NWIN = 7
MPAD = NWIN * WIN
PT = P // (NC * NS)
B = 896
BR = B // 128
NCHUNK = PT // B
TSLAB = WIN // NS
TR = TSLAB // 128
PR = P // 128


def _vox_body(x_ref, y_ref, z_ref, vox_ref):
    def quant(v):
        s = jnp.tanh(v / CONTRACT_SCALE)
        return jnp.clip(jnp.floor((s + 1.0) * 0.5 * RES), 0, RES - 1).astype(jnp.int32)

    qx = quant(x_ref[...])
    qy = quant(y_ref[...])
    qz = quant(z_ref[...])
    vox_ref[...] = (qx * RES + qy) * RES + qz


def _compute_vox(x2d, y2d, z2d):
    blk = (x2d.shape[0] // 21, x2d.shape[1])
    spec = pl.BlockSpec(blk, lambda i: (i, 0))
    return pl.pallas_call(
        _vox_body,
        grid=(21,),
        in_specs=[spec, spec, spec],
        out_specs=spec,
        out_shape=jax.ShapeDtypeStruct(x2d.shape, jnp.int32),
    )(x2d, y2d, z2d)


_TBLK = 4096


def _tr_body(f_ref, eye_ref, out_ref):
    t = lax.dot_general(eye_ref[...], f_ref[...],
                        (((1,), (1,)), ((), ())),
                        preferred_element_type=jnp.float32)
    ones = jnp.ones((1, _TBLK), jnp.float32)
    pad = jnp.zeros((F - C - 1, _TBLK), jnp.float32)
    out_ref[...] = jnp.concatenate([t, ones, pad], axis=0)


def _transpose_feats(feats, eye):
    return pl.pallas_call(
        _tr_body,
        grid=(P // _TBLK,),
        in_specs=[
            pl.BlockSpec((_TBLK, C), lambda i: (i, 0)),
            pl.BlockSpec((C, C), lambda i: (0, 0)),
        ],
        out_specs=pl.BlockSpec((F, _TBLK), lambda i: (0, i)),
        out_shape=jax.ShapeDtypeStruct((F, P), jnp.float32),
    )(feats, eye)


def _sc_body(vox_hbm, ft_hbm, zb_hbm, outT_hbm, *rest):
    vox_v, stage_v, vt_v, zb_v, ob_v = rest[:5]
    accs = rest[5:5 + NSUM]
    c = lax.axis_index("c")
    s = lax.axis_index("s")
    pbase = (c * NS + s) * PT
    iota = lax.iota(jnp.int32, 16)
    sbase = s * TSLAB

    pltpu.sync_copy(zb_hbm, zb_v)

    def ramp(base):
        def rbody(i, _):
            stage_v[0, pl.ds(i * 16, 16)] = iota + (base + i * 16)
            return 0

        lax.fori_loop(0, 8, rbody, 0)

    def window_body(j, _):
        lo = j * WIN

        def zero_body(t, _):
            ramp(sbase + t * 128)
            idx = stage_v.at[0]
            for f in range(NSUM):
                pltpu.sync_copy(zb_v, accs[f].at[idx])
            return 0

        lax.fori_loop(0, TR, zero_body, 0)
        plsc.subcore_barrier()

        def chunk_body(k, _):
            coff = pl.multiple_of(pbase + k * B, 8)
            pltpu.sync_copy(vox_hbm.at[pl.ds(coff, B)], vox_v)
            pltpu.sync_copy(ft_hbm.at[:, pl.ds(coff, B)], vt_v)

            def sub_body(r, _):
                def lane_body(i, _):
                    v = vox_v[pl.ds(r * 128 + i * 16, 16)]
                    m = (v >= lo) & (v < lo + WIN)
                    g = WIN + ((iota + i * 16 + r) & (GARB - 1))
                    stage_v[0, pl.ds(i * 16, 16)] = jnp.where(m, v - lo, g)
                    return 0

                lax.fori_loop(0, 8, lane_body, 0)
                idx = stage_v.at[0]
                roff = pl.multiple_of(r * 128, 8)
                for f in range(NSUM):
                    pltpu.sync_copy(vt_v.at[f, pl.ds(roff, 128)],
                                    accs[f].at[idx], add=True)
                return 0

            lax.fori_loop(0, BR, sub_body, 0)
            return 0

        lax.fori_loop(0, NCHUNK, chunk_body, 0)
        plsc.subcore_barrier()

        def out_body(t, _):
            ramp(sbase + t * 128)
            idx = stage_v.at[0]
            ooff = pl.multiple_of(lo + sbase + t * 128, 8)
            for f in range(NSUM):
                pltpu.sync_copy(accs[f].at[idx], ob_v)
                pltpu.sync_copy(ob_v, outT_hbm.at[c, f, pl.ds(ooff, 128)])
            return 0

        lax.fori_loop(0, TR, out_body, 0)
        plsc.subcore_barrier()
        return 0

    lax.fori_loop(0, NWIN, window_body, 0)


def _sc_scatter(vox, ftT, zb):
    mesh = plsc.VectorSubcoreMesh(core_axis_name="c", subcore_axis_name="s")
    fn = pl.kernel(
        _sc_body,
        out_type=jax.ShapeDtypeStruct((NC, NSUM, MPAD), jnp.float32),
        mesh=mesh,
        scratch_types=[
            pltpu.VMEM((B,), jnp.int32),
            pltpu.VMEM((1, 128), jnp.int32),
            pltpu.VMEM((F, B), jnp.float32),
            pltpu.VMEM((128,), jnp.float32),
            pltpu.VMEM((128,), jnp.float32),
        ] + [pltpu.VMEM_SHARED((WIN + GARB,), jnp.float32) for _ in range(NSUM)]
        + [pltpu.SemaphoreType.DMA],
    )
    return fn(vox, ftT, zb)


_HBLK = 2048


def _head_body(t0_ref, t1_ref, w_ref, b_ref, out_ref):
    tt = t0_ref[0] + t1_ref[0]
    sums = tt[:C, :]
    cnt = tt[C:C + 1, :]
    pooledT = sums / jnp.maximum(cnt, 1.0)
    y = lax.dot_general(pooledT, w_ref[...],
                        (((0,), (0,)), ((), ())),
                        preferred_element_type=jnp.float32)
    y = y + b_ref[...]
    col = lax.broadcasted_iota(jnp.int32, y.shape, 1)
    cval = jnp.where(col == C - 1, DENSITY_CLAMP, 5.0)
    out_ref[...] = jnp.tanh(y / cval) * cval


def _head(outT, w, b2d):
    return pl.pallas_call(
        _head_body,
        grid=(M // _HBLK,),
        in_specs=[
            pl.BlockSpec((1, NSUM, _HBLK), lambda i: (0, 0, i)),
            pl.BlockSpec((1, NSUM, _HBLK), lambda i: (1, 0, i)),
            pl.BlockSpec((C, C), lambda i: (0, 0)),
            pl.BlockSpec((1, C), lambda i: (0, 0)),
        ],
        out_specs=pl.BlockSpec((_HBLK, C), lambda i: (i, 0)),
        out_shape=jax.ShapeDtypeStruct((M, C), jnp.float32),
    )(outT, outT, w, b2d)


def kernel(feat_coord_in_world_frame, encoded_scene, W, b):
    x2d = feat_coord_in_world_frame[:, 0].reshape(P // 512, 512)
    y2d = feat_coord_in_world_frame[:, 1].reshape(P // 512, 512)
    z2d = feat_coord_in_world_frame[:, 2].reshape(P // 512, 512)
    vox = _compute_vox(x2d, y2d, z2d).reshape(P)

    ftT = _transpose_feats(encoded_scene, jnp.eye(C, dtype=jnp.float32))
    zb = jnp.zeros((128,), jnp.float32)
    outT = _sc_scatter(vox, ftT, zb)

    return _head(outT, W, b.reshape(1, C))
